# baseline (speedup 1.0000x reference)
import jax
import jax.numpy as jnp
from jax import lax
from jax.experimental import pallas as pl
from jax.experimental.pallas import tpu as pltpu
from jax.experimental.pallas import tpu_sc as plsc

VOCAB = 1000000
EMBD = 64
B = 4096
L = 50

NW = 32
BPW = B // NW
LANES = 16
NCH = VOCAB // 128
TAIL = VOCAB - NCH * 128


def _wid():
    return lax.axis_index("s") * 2 + lax.axis_index("c")


def _tr_body(tabt_hbm, out_hbm, in_v, out_v, tin_v, tout_v):
    wid = _wid()
    iotas = [lax.iota(jnp.int32, LANES) + (g * LANES) for g in range(4)]

    @pl.loop(wid, NCH, step=NW)
    def _(c):
        pltpu.sync_copy(tabt_hbm.at[:, pl.ds(c * 128, 128)], in_v)
        for r in range(128):
            rr = jnp.full((LANES,), r, jnp.int32)
            for g in range(4):
                out_v[r, pl.ds(g * LANES, LANES)] = plsc.load_gather(
                    in_v, [iotas[g], rr])
        pltpu.sync_copy(out_v, out_hbm.at[pl.ds(c * 128, 128)])

    @pl.when(wid == NCH % NW)
    def _():
        pltpu.sync_copy(tabt_hbm.at[:, pl.ds(NCH * 128, TAIL)], tin_v)
        for r in range(TAIL):
            rr = jnp.full((LANES,), r, jnp.int32)
            for g in range(4):
                tout_v[r, pl.ds(g * LANES, LANES)] = plsc.load_gather(
                    tin_v, [iotas[g], rr])
        pltpu.sync_copy(tout_v, out_hbm.at[pl.ds(NCH * 128, TAIL)])


def _emb_body(xt_hbm, tab_hbm, out_hbm, idx_v, rows_v, gsem, wsem):
    wid = _wid()
    b0 = wid * BPW

    pltpu.sync_copy(xt_hbm.at[:, pl.ds(b0, BPW)], idx_v)

    def gather(l, p):
        pltpu.async_copy(tab_hbm.at[idx_v.at[l]], rows_v.at[p], gsem.at[p])

    def gather_wait(l, p):
        pltpu.make_async_copy(tab_hbm.at[idx_v.at[l]], rows_v.at[p],
                              gsem.at[p]).wait()

    def write(l, p):
        pltpu.async_copy(rows_v.at[p], out_hbm.at[pl.ds(b0, BPW), l, :],
                         wsem.at[p])

    def write_wait(l, p):
        pltpu.make_async_copy(rows_v.at[p], out_hbm.at[pl.ds(b0, BPW), l, :],
                              wsem.at[p]).wait()

    gather(0, 0)

    @pl.loop(0, L // 2)
    def _(ll):
        for p in range(2):
            l = 2 * ll + p

            gather_wait(l, p)

            def prefetch(l=l, p=p):
                def drain(l=l, p=p):
                    write_wait(l - 1, 1 - p)

                pl.when(l >= 1)(drain)
                gather(l + 1, 1 - p)

            if p == 0:
                prefetch()
            else:
                pl.when(ll < L // 2 - 1)(prefetch)

            write(l, p)

    write_wait(L - 2, 0)
    write_wait(L - 1, 1)


@jax.jit
def _emb(xt, tabt):
    mesh = plsc.VectorSubcoreMesh(core_axis_name="c", subcore_axis_name="s")
    params = pltpu.CompilerParams(use_tc_tiling_on_sc=True,
                                  needs_layout_passes=False)
    ktr = pl.kernel(
        _tr_body,
        out_type=jax.ShapeDtypeStruct((VOCAB, 128), jnp.float32),
        mesh=mesh,
        compiler_params=params,
        scratch_types=[
            pltpu.VMEM((EMBD, 128), jnp.float32),
            pltpu.VMEM((128, 128), jnp.float32),
            pltpu.VMEM((EMBD, TAIL), jnp.float32),
            pltpu.VMEM((TAIL, 128), jnp.float32),
        ],
    )
    kg = pl.kernel(
        _emb_body,
        out_type=jax.ShapeDtypeStruct((B, L, 128), jnp.float32),
        mesh=mesh,
        compiler_params=params,
        scratch_types=[
            pltpu.VMEM((L, BPW), jnp.int32),
            pltpu.VMEM((2, BPW, 128), jnp.float32),
            pltpu.SemaphoreType.DMA((2,)),
            pltpu.SemaphoreType.DMA((2,)),
        ],
    )
    return kg(xt, ktr(tabt))


def kernel(x, table):
    xt = jnp.transpose(x.astype(jnp.int32))
    tabt = jnp.transpose(table)
    return _emb(xt, tabt)[:, :, 0:EMBD]


# speedup vs baseline: 1.6942x; 1.6942x over previous
import jax
import jax.numpy as jnp
from jax import lax
from jax.experimental import pallas as pl
from jax.experimental.pallas import tpu as pltpu
from jax.experimental.pallas import tpu_sc as plsc

VOCAB = 1000000
EMBD = 64
B = 4096
L = 50

NW = 32
BPW = B // NW
LANES = 16
NCH = VOCAB // 128
TAIL = VOCAB - NCH * 128


def _wid():
    return lax.axis_index("s") * 2 + lax.axis_index("c")


def _tr_body(tabt_hbm, out_hbm, in_v, out_v, tin_v, tout_v, isem, osem):
    wid = _wid()
    iotas = [lax.iota(jnp.int32, LANES) + (g * LANES) for g in range(4)]

    def transpose(src, dst, nrows):
        for r4 in range(0, nrows, 4):
            vals = []
            for r in range(r4, r4 + 4):
                rr = jnp.full((LANES,), r, jnp.int32)
                for g in range(4):
                    vals.append((r, g, plsc.load_gather(src, [iotas[g], rr])))
            for r, g, v in vals:
                dst[r, pl.ds(g * LANES, LANES)] = v

    def in_copy(c, p):
        pltpu.async_copy(tabt_hbm.at[:, pl.ds(c * 128, 128)], in_v.at[p],
                         isem.at[p])

    def in_wait(c, p):
        pltpu.make_async_copy(tabt_hbm.at[:, pl.ds(c * 128, 128)],
                              in_v.at[p], isem.at[p]).wait()

    def out_write(c, p):
        pltpu.async_copy(out_v.at[p], out_hbm.at[pl.ds(c * 128, 128)],
                         osem.at[p])

    def out_wait(c, p):
        pltpu.make_async_copy(out_v.at[p], out_hbm.at[pl.ds(c * 128, 128)],
                              osem.at[p]).wait()

    in_copy(wid, 0)

    @pl.loop(0, 123)
    def _(ii):
        for p in range(2):
            i = 2 * ii + p
            c = wid + i * NW

            @pl.when(c < NCH)
            def _(c=c, p=p, i=i, ii=ii):
                in_wait(c, p)

                @pl.when(c + NW < NCH)
                def _(c=c, p=p):
                    in_copy(c + NW, 1 - p)

                def drain(c=c, p=p):
                    out_wait(c - 2 * NW, p)

                if p == 0:
                    pl.when(ii >= 1)(drain)
                else:
                    pl.when(i >= 2)(drain)

                transpose(in_v.at[p], out_v.at[p], 128)
                out_write(c, p)

    out_wait(wid + 243 * NW, 1)

    @pl.when(wid + 244 * NW < NCH)
    def _():
        out_wait(wid + 244 * NW, 0)

    @pl.when(wid == NCH % NW)
    def _():
        pltpu.sync_copy(tabt_hbm.at[:, pl.ds(NCH * 128, TAIL)], tin_v)
        transpose(tin_v, tout_v, TAIL)
        pltpu.sync_copy(tout_v, out_hbm.at[pl.ds(NCH * 128, TAIL)])


def _emb_body(xt_hbm, tab_hbm, out_hbm, idx_v, rows_v, gsem, wsem):
    wid = _wid()
    b0 = wid * BPW

    pltpu.sync_copy(xt_hbm.at[:, pl.ds(b0, BPW)], idx_v)

    def gather(l, p):
        pltpu.async_copy(tab_hbm.at[idx_v.at[l]], rows_v.at[p], gsem.at[p])

    def gather_wait(l, p):
        pltpu.make_async_copy(tab_hbm.at[idx_v.at[l]], rows_v.at[p],
                              gsem.at[p]).wait()

    def write(l, p):
        pltpu.async_copy(rows_v.at[p], out_hbm.at[pl.ds(b0, BPW), l, :],
                         wsem.at[p])

    def write_wait(l, p):
        pltpu.make_async_copy(rows_v.at[p], out_hbm.at[pl.ds(b0, BPW), l, :],
                              wsem.at[p]).wait()

    gather(0, 0)

    @pl.loop(0, L // 2)
    def _(ll):
        for p in range(2):
            l = 2 * ll + p

            gather_wait(l, p)

            def prefetch(l=l, p=p):
                def drain(l=l, p=p):
                    write_wait(l - 1, 1 - p)

                pl.when(l >= 1)(drain)
                gather(l + 1, 1 - p)

            if p == 0:
                prefetch()
            else:
                pl.when(ll < L // 2 - 1)(prefetch)

            write(l, p)

    write_wait(L - 2, 0)
    write_wait(L - 1, 1)


@jax.jit
def _emb(xt, tabt):
    mesh = plsc.VectorSubcoreMesh(core_axis_name="c", subcore_axis_name="s")
    params = pltpu.CompilerParams(use_tc_tiling_on_sc=True,
                                  needs_layout_passes=False)
    ktr = pl.kernel(
        _tr_body,
        out_type=jax.ShapeDtypeStruct((VOCAB, 128), jnp.float32),
        mesh=mesh,
        compiler_params=params,
        scratch_types=[
            pltpu.VMEM((2, EMBD, 128), jnp.float32),
            pltpu.VMEM((2, 128, 128), jnp.float32),
            pltpu.VMEM((EMBD, TAIL), jnp.float32),
            pltpu.VMEM((TAIL, 128), jnp.float32),
            pltpu.SemaphoreType.DMA((2,)),
            pltpu.SemaphoreType.DMA((2,)),
        ],
    )
    kg = pl.kernel(
        _emb_body,
        out_type=jax.ShapeDtypeStruct((B, L, 128), jnp.float32),
        mesh=mesh,
        compiler_params=params,
        scratch_types=[
            pltpu.VMEM((L, BPW), jnp.int32),
            pltpu.VMEM((2, BPW, 128), jnp.float32),
            pltpu.SemaphoreType.DMA((2,)),
            pltpu.SemaphoreType.DMA((2,)),
        ],
    )
    return kg(xt, ktr(tabt))


def kernel(x, table):
    xt = jnp.transpose(x.astype(jnp.int32))
    tabt = jnp.transpose(table)
    return _emb(xt, tabt)[:, :, 0:EMBD]


# speedup vs baseline: 3.1269x; 1.8457x over previous
import jax
import jax.numpy as jnp
from jax import lax
from jax.experimental import pallas as pl
from jax.experimental.pallas import tpu as pltpu
from jax.experimental.pallas import tpu_sc as plsc

VOCAB = 1000000
EMBD = 64
B = 4096
L = 50

NW = 32
BPW = B // NW


def _emb_body(xt_hbm, tab_hbm, out_hbm, idx_v, rows_v, gsem, wsem):
    nc = 2
    wid = lax.axis_index("s") * nc + lax.axis_index("c")
    b0 = wid * BPW

    pltpu.sync_copy(xt_hbm.at[:, pl.ds(b0, BPW)], idx_v)

    def gather(l, p):
        pltpu.async_copy(tab_hbm.at[idx_v.at[l]], rows_v.at[p], gsem.at[p])

    def gather_wait(l, p):
        pltpu.make_async_copy(tab_hbm.at[idx_v.at[l]], rows_v.at[p],
                              gsem.at[p]).wait()

    def write(l, p):
        pltpu.async_copy(rows_v.at[p], out_hbm.at[pl.ds(b0, BPW), l, :],
                         wsem.at[p])

    def write_wait(l, p):
        pltpu.make_async_copy(rows_v.at[p], out_hbm.at[pl.ds(b0, BPW), l, :],
                              wsem.at[p]).wait()

    gather(0, 0)

    @pl.loop(0, L // 2)
    def _(ll):
        for p in range(2):
            l = 2 * ll + p

            gather_wait(l, p)

            def prefetch(l=l, p=p):
                def drain(l=l, p=p):
                    write_wait(l - 1, 1 - p)

                pl.when(l >= 1)(drain)
                gather(l + 1, 1 - p)

            if p == 0:
                prefetch()
            else:
                pl.when(ll < L // 2 - 1)(prefetch)

            write(l, p)

    write_wait(L - 2, 0)
    write_wait(L - 1, 1)


@jax.jit
def _emb(xt, tab_pad):
    mesh = plsc.VectorSubcoreMesh(core_axis_name="c", subcore_axis_name="s")
    f = pl.kernel(
        _emb_body,
        out_type=jax.ShapeDtypeStruct((B, L, 128), jnp.float32),
        mesh=mesh,
        compiler_params=pltpu.CompilerParams(use_tc_tiling_on_sc=True,
                                             needs_layout_passes=False),
        scratch_types=[
            pltpu.VMEM((L, BPW), jnp.int32),
            pltpu.VMEM((2, BPW, 128), jnp.float32),
            pltpu.SemaphoreType.DMA((2,)),
            pltpu.SemaphoreType.DMA((2,)),
        ],
    )
    return f(xt, tab_pad)


def kernel(x, table):
    xt = jnp.transpose(x.astype(jnp.int32))
    tab_pad = jnp.pad(table, ((0, 0), (0, 128 - EMBD)))
    return _emb(xt, tab_pad)[:, :, 0:EMBD]


# speedup vs baseline: 4.3866x; 1.4029x over previous
import jax
import jax.numpy as jnp
from jax import lax
from jax.experimental import pallas as pl
from jax.experimental.pallas import tpu as pltpu
from jax.experimental.pallas import tpu_sc as plsc

VOCAB = 1000000
EMBD = 64
B = 4096
L = 50

NW = 32
BPW = B // NW
LANES = 16
NCH = VOCAB // 128
TAIL = VOCAB - NCH * 128


def _wid():
    return lax.axis_index("s") * 2 + lax.axis_index("c")


def _tr_body(tabt_hbm, out_hbm, in_v, out_v, tin_v, tout_v, isem, osem):
    wid = _wid()
    iota = lax.iota(jnp.int32, LANES)
    iotas = [iota + (g * LANES) for g in range(4)]

    def transpose(src, dst, ncol):
        @pl.loop(0, LANES)
        def _(d):
            rot = lax.rem(iota + d, LANES)
            for g in range(4):
                ccs, vals = [], []
                for k in range(ncol // LANES):
                    cc = rot + (k * LANES)
                    ccs.append(cc)
                    vals.append(plsc.load_gather(src, [iotas[g], cc]))
                for cc, v in zip(ccs, vals):
                    plsc.store_scatter(dst, [cc, iotas[g]], v)

    def in_copy(c, p):
        pltpu.async_copy(tabt_hbm.at[:, pl.ds(c * 128, 128)], in_v.at[p],
                         isem.at[p])

    def in_wait(c, p):
        pltpu.make_async_copy(tabt_hbm.at[:, pl.ds(c * 128, 128)],
                              in_v.at[p], isem.at[p]).wait()

    def out_write(c, p):
        pltpu.async_copy(out_v.at[p], out_hbm.at[pl.ds(c * 128, 128)],
                         osem.at[p])

    def out_wait(c, p):
        pltpu.make_async_copy(out_v.at[p], out_hbm.at[pl.ds(c * 128, 128)],
                              osem.at[p]).wait()

    in_copy(wid, 0)

    @pl.loop(0, 123)
    def _(ii):
        for p in range(2):
            i = 2 * ii + p
            c = wid + i * NW

            @pl.when(c < NCH)
            def _(c=c, p=p, i=i, ii=ii):
                in_wait(c, p)

                @pl.when(c + NW < NCH)
                def _(c=c, p=p):
                    in_copy(c + NW, 1 - p)

                def drain(c=c, p=p):
                    out_wait(c - 2 * NW, p)

                if p == 0:
                    pl.when(ii >= 1)(drain)
                else:
                    pl.when(i >= 2)(drain)

                transpose(in_v.at[p], out_v.at[p], 128)
                out_write(c, p)

    out_wait(wid + 243 * NW, 1)

    @pl.when(wid + 244 * NW < NCH)
    def _():
        out_wait(wid + 244 * NW, 0)

    @pl.when(wid == NCH % NW)
    def _():
        pltpu.sync_copy(tabt_hbm.at[:, pl.ds(NCH * 128, TAIL)], tin_v)
        transpose(tin_v, tout_v, TAIL)
        pltpu.sync_copy(tout_v, out_hbm.at[pl.ds(NCH * 128, TAIL)])


def _emb_body(xt_hbm, tab_hbm, out_hbm, idx_v, rows_v, gsem, wsem):
    wid = _wid()
    b0 = wid * BPW

    pltpu.sync_copy(xt_hbm.at[:, pl.ds(b0, BPW)], idx_v)

    def gather(l, p):
        pltpu.async_copy(tab_hbm.at[idx_v.at[l]], rows_v.at[p], gsem.at[p])

    def gather_wait(l, p):
        pltpu.make_async_copy(tab_hbm.at[idx_v.at[l]], rows_v.at[p],
                              gsem.at[p]).wait()

    def write(l, p):
        pltpu.async_copy(rows_v.at[p], out_hbm.at[pl.ds(b0, BPW), l, :],
                         wsem.at[p])

    def write_wait(l, p):
        pltpu.make_async_copy(rows_v.at[p], out_hbm.at[pl.ds(b0, BPW), l, :],
                              wsem.at[p]).wait()

    gather(0, 0)

    @pl.loop(0, L // 2)
    def _(ll):
        for p in range(2):
            l = 2 * ll + p

            gather_wait(l, p)

            def prefetch(l=l, p=p):
                def drain(l=l, p=p):
                    write_wait(l - 1, 1 - p)

                pl.when(l >= 1)(drain)
                gather(l + 1, 1 - p)

            if p == 0:
                prefetch()
            else:
                pl.when(ll < L // 2 - 1)(prefetch)

            write(l, p)

    write_wait(L - 2, 0)
    write_wait(L - 1, 1)


@jax.jit
def _emb(xt, tabt):
    mesh = plsc.VectorSubcoreMesh(core_axis_name="c", subcore_axis_name="s")
    params = pltpu.CompilerParams(use_tc_tiling_on_sc=True,
                                  needs_layout_passes=False)
    ktr = pl.kernel(
        _tr_body,
        out_type=jax.ShapeDtypeStruct((VOCAB, 128), jnp.float32),
        mesh=mesh,
        compiler_params=params,
        scratch_types=[
            pltpu.VMEM((2, EMBD, 128), jnp.float32),
            pltpu.VMEM((2, 128, 128), jnp.float32),
            pltpu.VMEM((EMBD, TAIL), jnp.float32),
            pltpu.VMEM((TAIL, 128), jnp.float32),
            pltpu.SemaphoreType.DMA((2,)),
            pltpu.SemaphoreType.DMA((2,)),
        ],
    )
    kg = pl.kernel(
        _emb_body,
        out_type=jax.ShapeDtypeStruct((B, L, 128), jnp.float32),
        mesh=mesh,
        compiler_params=params,
        scratch_types=[
            pltpu.VMEM((L, BPW), jnp.int32),
            pltpu.VMEM((2, BPW, 128), jnp.float32),
            pltpu.SemaphoreType.DMA((2,)),
            pltpu.SemaphoreType.DMA((2,)),
        ],
    )
    return kg(xt, ktr(tabt))


def kernel(x, table):
    xt = jnp.transpose(x.astype(jnp.int32))
    tabt = jnp.transpose(table)
    return _emb(xt, tabt)[:, :, 0:EMBD]
